# baseline (speedup 1.0000x reference)
import jax
import jax.numpy as jnp
from jax.experimental import pallas as pl
from jax.experimental.pallas import tpu as pltpu

_B = 4
_D = 256
_K = 1024
_THW = 8 * 32 * 32
_NB = 4096
_NBLK = _THW // _NB
_N = _B * _THW
_COMMIT = 0.25


def _vq_body(ek2_ref, e_ref, et_ref, z_ref,
             zq_ref, idx_ref, loss_ref, perp_ref,
             ssd_acc, cnt_acc):
    b = pl.program_id(0)
    j = pl.program_id(1)
    first = jnp.logical_and(b == 0, j == 0)
    last = jnp.logical_and(b == pl.num_programs(0) - 1,
                           j == pl.num_programs(1) - 1)

    @pl.when(first)
    def _init():
        ssd_acc[0, 0] = 0.0
        cnt_acc[...] = jnp.zeros_like(cnt_acc)

    z_blk = z_ref[0]
    sm2 = jnp.dot(e_ref[...], z_blk,
                  preferred_element_type=jnp.float32)
    zn2 = jnp.sum(z_blk * z_blk, axis=0, keepdims=True)
    d = (zn2 + ek2_ref[...]) + sm2

    m = jnp.min(d, axis=0, keepdims=True)
    iota = jax.lax.broadcasted_iota(jnp.int32, (_K, _NB), 0)
    idx = jnp.min(jnp.where(d == m, iota, _K),
                  axis=0, keepdims=True)
    idx_ref[0] = idx

    onehot = (iota == idx).astype(jnp.float32)
    zq = jnp.dot(et_ref[...], onehot,
                 preferred_element_type=jnp.float32)
    diff = zq - z_blk
    zq_ref[0] = z_blk + diff
    ssd_acc[0, 0] += jnp.sum(diff * diff)
    cnt_acc[...] += jnp.sum(onehot, axis=1, keepdims=True)

    @pl.when(last)
    def _fini():
        loss = (1.0 + _COMMIT) * ssd_acc[0, 0] / float(_N * _D)
        loss_ref[...] = jnp.reshape(loss, (1, 1))
        p = cnt_acc[...] * (1.0 / float(_N))
        perp = jnp.exp(-jnp.sum(p * jnp.log(p + 1e-10)))
        perp_ref[...] = jnp.reshape(perp, (1, 1))


def kernel(z, embedding):
    z3 = z.reshape(_B, _D, _THW)
    ek2 = (embedding ** 2).sum(axis=1).reshape(_K, 1)
    em2 = -2.0 * embedding
    et = embedding.T

    grid = (_B, _NBLK)
    zq3, idx3, loss, perp = pl.pallas_call(
        _vq_body,
        grid=grid,
        in_specs=[
            pl.BlockSpec((_K, 1), lambda b, j: (0, 0)),
            pl.BlockSpec((_K, _D), lambda b, j: (0, 0)),
            pl.BlockSpec((_D, _K), lambda b, j: (0, 0)),
            pl.BlockSpec((1, _D, _NB), lambda b, j: (b, 0, j)),
        ],
        out_specs=[
            pl.BlockSpec((1, _D, _NB), lambda b, j: (b, 0, j)),
            pl.BlockSpec((1, 1, _NB), lambda b, j: (b * _NBLK + j, 0, 0)),
            pl.BlockSpec((1, 1), lambda b, j: (0, 0)),
            pl.BlockSpec((1, 1), lambda b, j: (0, 0)),
        ],
        out_shape=[
            jax.ShapeDtypeStruct((_B, _D, _THW), jnp.float32),
            jax.ShapeDtypeStruct((_B * _NBLK, 1, _NB), jnp.int32),
            jax.ShapeDtypeStruct((1, 1), jnp.float32),
            jax.ShapeDtypeStruct((1, 1), jnp.float32),
        ],
        scratch_shapes=[
            pltpu.SMEM((1, 1), jnp.float32),
            pltpu.VMEM((_K, 1), jnp.float32),
        ],
        compiler_params=pltpu.CompilerParams(
            dimension_semantics=("arbitrary", "arbitrary"),
        ),
    )(ek2, em2, et, z3)

    z_q = zq3.reshape(z.shape)
    idx = idx3.reshape(_N)
    return (z_q, loss[0, 0], idx, perp[0, 0])


# speedup vs baseline: 1.0131x; 1.0131x over previous
import jax
import jax.numpy as jnp
from jax.experimental import pallas as pl
from jax.experimental.pallas import tpu as pltpu

_B = 4
_D = 256
_K = 1024
_THW = 8 * 32 * 32
_NB = 2048
_NBLK = _THW // _NB
_N = _B * _THW
_COMMIT = 0.25


def _vq_body(ek2_ref, e_ref, et_ref, z_ref,
             zq_ref, idx_ref, loss_ref, perp_ref,
             ssd_acc, cnt_acc):
    b = pl.program_id(0)
    j = pl.program_id(1)
    first = jnp.logical_and(b == 0, j == 0)
    last = jnp.logical_and(b == pl.num_programs(0) - 1,
                           j == pl.num_programs(1) - 1)

    @pl.when(first)
    def _init():
        ssd_acc[0, 0] = 0.0
        cnt_acc[...] = jnp.zeros_like(cnt_acc)

    z_blk = z_ref[0]
    sm2 = jnp.dot(e_ref[...], z_blk,
                  preferred_element_type=jnp.float32)
    zn2 = jnp.sum(z_blk * z_blk, axis=0, keepdims=True)
    d = (zn2 + ek2_ref[...]) + sm2

    m = jnp.min(d, axis=0, keepdims=True)
    iota = jax.lax.broadcasted_iota(jnp.int32, (_K, _NB), 0)
    idx = jnp.min(jnp.where(d == m, iota, _K),
                  axis=0, keepdims=True)
    idx_ref[0] = idx

    onehot = (iota == idx).astype(jnp.float32)
    zq = jnp.dot(et_ref[...], onehot,
                 preferred_element_type=jnp.float32)
    diff = zq - z_blk
    zq_ref[0] = z_blk + diff
    ssd_acc[0, 0] += jnp.sum(diff * diff)
    cnt_acc[...] += jnp.sum(onehot, axis=1, keepdims=True)

    @pl.when(last)
    def _fini():
        loss = (1.0 + _COMMIT) * ssd_acc[0, 0] / float(_N * _D)
        loss_ref[...] = jnp.reshape(loss, (1, 1))
        p = cnt_acc[...] * (1.0 / float(_N))
        perp = jnp.exp(-jnp.sum(p * jnp.log(p + 1e-10)))
        perp_ref[...] = jnp.reshape(perp, (1, 1))


def kernel(z, embedding):
    z3 = z.reshape(_B, _D, _THW)
    ek2 = (embedding ** 2).sum(axis=1).reshape(_K, 1)
    em2 = -2.0 * embedding
    et = embedding.T

    grid = (_B, _NBLK)
    zq3, idx3, loss, perp = pl.pallas_call(
        _vq_body,
        grid=grid,
        in_specs=[
            pl.BlockSpec((_K, 1), lambda b, j: (0, 0)),
            pl.BlockSpec((_K, _D), lambda b, j: (0, 0)),
            pl.BlockSpec((_D, _K), lambda b, j: (0, 0)),
            pl.BlockSpec((1, _D, _NB), lambda b, j: (b, 0, j)),
        ],
        out_specs=[
            pl.BlockSpec((1, _D, _NB), lambda b, j: (b, 0, j)),
            pl.BlockSpec((1, 1, _NB), lambda b, j: (b * _NBLK + j, 0, 0)),
            pl.BlockSpec((1, 1), lambda b, j: (0, 0)),
            pl.BlockSpec((1, 1), lambda b, j: (0, 0)),
        ],
        out_shape=[
            jax.ShapeDtypeStruct((_B, _D, _THW), jnp.float32),
            jax.ShapeDtypeStruct((_B * _NBLK, 1, _NB), jnp.int32),
            jax.ShapeDtypeStruct((1, 1), jnp.float32),
            jax.ShapeDtypeStruct((1, 1), jnp.float32),
        ],
        scratch_shapes=[
            pltpu.SMEM((1, 1), jnp.float32),
            pltpu.VMEM((_K, 1), jnp.float32),
        ],
        compiler_params=pltpu.CompilerParams(
            dimension_semantics=("arbitrary", "arbitrary"),
        ),
    )(ek2, em2, et, z3)

    z_q = zq3.reshape(z.shape)
    idx = idx3.reshape(_N)
    return (z_q, loss[0, 0], idx, perp[0, 0])


# speedup vs baseline: 1.2622x; 1.2459x over previous
import jax
import jax.numpy as jnp
from jax.experimental import pallas as pl
from jax.experimental.pallas import tpu as pltpu

_B = 4
_D = 256
_K = 1024
_THW = 8 * 32 * 32
_NB = 2048
_NBLK = _THW // _NB
_N = _B * _THW
_COMMIT = 0.25


def _vq_body(ek2_ref, e_ref, et_ref, z_ref,
             zq_ref, idx_ref, loss_ref, perp_ref,
             ssd_acc, cnt_acc):
    b = pl.program_id(0)
    j = pl.program_id(1)
    first = jnp.logical_and(b == 0, j == 0)
    last = jnp.logical_and(b == pl.num_programs(0) - 1,
                           j == pl.num_programs(1) - 1)

    @pl.when(first)
    def _init():
        ssd_acc[0, 0] = 0.0
        cnt_acc[...] = jnp.zeros_like(cnt_acc)

    z_blk = z_ref[0]
    sm2 = jnp.dot(e_ref[...], z_blk,
                  preferred_element_type=jnp.float32)
    zn2 = jnp.sum(z_blk * z_blk, axis=0, keepdims=True)
    d = (zn2 + ek2_ref[...]) + sm2

    m = jnp.min(d, axis=0, keepdims=True)
    iota = jax.lax.broadcasted_iota(jnp.int32, (_K, _NB), 0)
    idx = jnp.min(jnp.where(d == m, iota, _K),
                  axis=0, keepdims=True)
    idx_ref[0] = idx

    onehot = (iota == idx).astype(jnp.float32)
    zq = jnp.dot(et_ref[...], onehot,
                 preferred_element_type=jnp.float32)
    diff = zq - z_blk
    zq_ref[0] = z_blk + diff
    ssd_acc[0, 0] += jnp.sum(diff * diff)
    cnt_acc[...] += jnp.sum(onehot, axis=1, keepdims=True)

    @pl.when(last)
    def _fini():
        loss = (1.0 + _COMMIT) * ssd_acc[0, 0] / float(_N * _D)
        loss_ref[...] = jnp.reshape(loss, (1, 1))
        p = cnt_acc[...] * (1.0 / float(_N))
        perp = jnp.exp(-jnp.sum(p * jnp.log(p + 1e-10)))
        perp_ref[...] = jnp.reshape(perp, (1, 1))


def kernel(z, embedding):
    z3 = z.reshape(_B, _D, _THW)
    ek2 = (embedding ** 2).sum(axis=1).reshape(_K, 1)
    em2 = -2.0 * embedding
    et = embedding.T

    grid = (_B, _NBLK)
    zq3, idx3, loss, perp = pl.pallas_call(
        _vq_body,
        grid=grid,
        in_specs=[
            pl.BlockSpec((_K, 1), lambda b, j: (0, 0)),
            pl.BlockSpec((_K, _D), lambda b, j: (0, 0)),
            pl.BlockSpec((_D, _K), lambda b, j: (0, 0)),
            pl.BlockSpec((1, _D, _NB), lambda b, j: (b, 0, j)),
        ],
        out_specs=[
            pl.BlockSpec((1, _D, _NB), lambda b, j: (b, 0, j)),
            pl.BlockSpec((1, 1, _NB), lambda b, j: (b * _NBLK + j, 0, 0)),
            pl.BlockSpec((1, 1), lambda b, j: (0, 0)),
            pl.BlockSpec((1, 1), lambda b, j: (0, 0)),
        ],
        out_shape=[
            jax.ShapeDtypeStruct((_B, _D, _THW), jnp.float32),
            jax.ShapeDtypeStruct((_B * _NBLK, 1, _NB), jnp.int32),
            jax.ShapeDtypeStruct((1, 1), jnp.float32),
            jax.ShapeDtypeStruct((1, 1), jnp.float32),
        ],
        scratch_shapes=[
            pltpu.SMEM((1, 1), jnp.float32),
            pltpu.VMEM((_K, 1), jnp.float32),
        ],
        compiler_params=pltpu.CompilerParams(
            dimension_semantics=("arbitrary", "arbitrary"),
        ),
    )(ek2, em2, et, z3)

    return (zq3, loss, idx3, perp)


# speedup vs baseline: 1.3237x; 1.0487x over previous
import jax
import jax.numpy as jnp
from jax.experimental import pallas as pl
from jax.experimental.pallas import tpu as pltpu

_B = 4
_D = 256
_K = 1024
_THW = 8 * 32 * 32
_N = _B * _THW
_R = 2048
_G = _N // _R
_COMMIT = 0.25


def _vq_body(ek2_ref, em2t_ref, e_ref, z_ref,
             zq_ref, idx_ref, loss_ref, perp_ref,
             ssd_acc, cnt_acc):
    g = pl.program_id(0)
    first = g == 0
    last = g == _G - 1

    @pl.when(first)
    def _init():
        ssd_acc[0, 0] = 0.0
        cnt_acc[...] = jnp.zeros_like(cnt_acc)

    z_blk = z_ref[...]
    s2 = jnp.dot(z_blk, em2t_ref[...],
                 preferred_element_type=jnp.float32)
    zn2 = jnp.sum(z_blk * z_blk, axis=1, keepdims=True)
    d = (zn2 + ek2_ref[...]) + s2

    m = jnp.min(d, axis=1, keepdims=True)
    iota = jax.lax.broadcasted_iota(jnp.int32, (_R, _K), 1)
    idx = jnp.min(jnp.where(d == m, iota, _K),
                  axis=1, keepdims=True)
    idx_ref[...] = idx

    onehot = (iota == idx).astype(jnp.float32)
    zq = jnp.dot(onehot, e_ref[...],
                 preferred_element_type=jnp.float32)
    diff = zq - z_blk
    zq_ref[...] = z_blk + diff
    ssd_acc[0, 0] += jnp.sum(diff * diff)
    cnt_acc[...] += jnp.sum(onehot, axis=0, keepdims=True)

    @pl.when(last)
    def _fini():
        loss = (1.0 + _COMMIT) * ssd_acc[0, 0] / float(_N * _D)
        loss_ref[...] = jnp.reshape(loss, (1, 1))
        p = cnt_acc[...] * (1.0 / float(_N))
        perp = jnp.exp(-jnp.sum(p * jnp.log(p + 1e-10)))
        perp_ref[...] = jnp.reshape(perp, (1, 1))


def kernel(z, embedding):
    z_flat = jnp.transpose(z, (0, 2, 3, 4, 1)).reshape(_N, _D)
    ek2 = (embedding ** 2).sum(axis=1).reshape(1, _K)
    em2t = -2.0 * embedding.T

    zq_flat, idx2, loss, perp = pl.pallas_call(
        _vq_body,
        grid=(_G,),
        in_specs=[
            pl.BlockSpec((1, _K), lambda g: (0, 0)),
            pl.BlockSpec((_D, _K), lambda g: (0, 0)),
            pl.BlockSpec((_K, _D), lambda g: (0, 0)),
            pl.BlockSpec((_R, _D), lambda g: (g, 0)),
        ],
        out_specs=[
            pl.BlockSpec((_R, _D), lambda g: (g, 0)),
            pl.BlockSpec((_R, 1), lambda g: (g, 0)),
            pl.BlockSpec((1, 1), lambda g: (0, 0)),
            pl.BlockSpec((1, 1), lambda g: (0, 0)),
        ],
        out_shape=[
            jax.ShapeDtypeStruct((_N, _D), jnp.float32),
            jax.ShapeDtypeStruct((_N, 1), jnp.int32),
            jax.ShapeDtypeStruct((1, 1), jnp.float32),
            jax.ShapeDtypeStruct((1, 1), jnp.float32),
        ],
        scratch_shapes=[
            pltpu.SMEM((1, 1), jnp.float32),
            pltpu.VMEM((1, _K), jnp.float32),
        ],
        compiler_params=pltpu.CompilerParams(
            dimension_semantics=("arbitrary",),
        ),
    )(ek2, em2t, embedding, z_flat)

    z_q = jnp.transpose(zq_flat.reshape(_B, 8, 32, 32, _D), (0, 4, 1, 2, 3))
    idx = idx2.reshape(_N)
    return (z_q, loss[0, 0], idx, perp[0, 0])


# speedup vs baseline: 1.5302x; 1.1560x over previous
import jax
import jax.numpy as jnp
from jax.experimental import pallas as pl
from jax.experimental.pallas import tpu as pltpu

_B = 4
_D = 256
_K = 1024
_THW = 8 * 32 * 32
_N = _B * _THW
_R = 2048
_G = _N // _R
_COMMIT = 0.25


def _vq_body(ek2_ref, em2t_ref, e_ref, z_ref,
             zq_ref, idx_ref, loss_ref, perp_ref,
             ssd_acc, cnt_acc):
    g = pl.program_id(0)
    first = g == 0
    last = g == _G - 1

    @pl.when(first)
    def _init():
        ssd_acc[0, 0] = 0.0
        cnt_acc[...] = jnp.zeros_like(cnt_acc)

    _H = _R // 2
    iota = jax.lax.broadcasted_iota(jnp.int32, (_H, _K), 1)
    ssd_parts = []
    cnt_parts = []
    for h in range(2):
        rows = pl.ds(h * _H, _H)
        z_blk = z_ref[rows, :]
        s2 = jnp.dot(z_blk, em2t_ref[...],
                     preferred_element_type=jnp.float32)
        zn2 = jnp.sum(z_blk * z_blk, axis=1, keepdims=True)
        d = (zn2 + ek2_ref[...]) + s2

        m = jnp.min(d, axis=1, keepdims=True)
        idx = jnp.min(jnp.where(d == m, iota, _K),
                      axis=1, keepdims=True)
        idx_ref[rows, :] = idx

        onehot = (iota == idx).astype(jnp.float32)
        zq = jnp.dot(onehot, e_ref[...],
                     preferred_element_type=jnp.float32)
        diff = zq - z_blk
        zq_ref[rows, :] = z_blk + diff
        ssd_parts.append(jnp.sum(diff * diff))
        cnt_parts.append(jnp.sum(onehot, axis=0, keepdims=True))
    ssd_acc[0, 0] += ssd_parts[0] + ssd_parts[1]
    cnt_acc[...] += cnt_parts[0] + cnt_parts[1]

    @pl.when(last)
    def _fini():
        loss = (1.0 + _COMMIT) * ssd_acc[0, 0] / float(_N * _D)
        loss_ref[...] = jnp.reshape(loss, (1, 1))
        p = cnt_acc[...] * (1.0 / float(_N))
        perp = jnp.exp(-jnp.sum(p * jnp.log(p + 1e-10)))
        perp_ref[...] = jnp.reshape(perp, (1, 1))


def kernel(z, embedding):
    z_flat = jnp.transpose(z, (0, 2, 3, 4, 1)).reshape(_N, _D)
    ek2 = (embedding ** 2).sum(axis=1).reshape(1, _K)
    em2t = -2.0 * embedding.T

    zq_flat, idx2, loss, perp = pl.pallas_call(
        _vq_body,
        grid=(_G,),
        in_specs=[
            pl.BlockSpec((1, _K), lambda g: (0, 0)),
            pl.BlockSpec((_D, _K), lambda g: (0, 0)),
            pl.BlockSpec((_K, _D), lambda g: (0, 0)),
            pl.BlockSpec((_R, _D), lambda g: (g, 0)),
        ],
        out_specs=[
            pl.BlockSpec((_R, _D), lambda g: (g, 0)),
            pl.BlockSpec((_R, 1), lambda g: (g, 0)),
            pl.BlockSpec((1, 1), lambda g: (0, 0)),
            pl.BlockSpec((1, 1), lambda g: (0, 0)),
        ],
        out_shape=[
            jax.ShapeDtypeStruct((_N, _D), jnp.float32),
            jax.ShapeDtypeStruct((_N, 1), jnp.int32),
            jax.ShapeDtypeStruct((1, 1), jnp.float32),
            jax.ShapeDtypeStruct((1, 1), jnp.float32),
        ],
        scratch_shapes=[
            pltpu.SMEM((1, 1), jnp.float32),
            pltpu.VMEM((1, _K), jnp.float32),
        ],
        compiler_params=pltpu.CompilerParams(
            dimension_semantics=("arbitrary",),
        ),
    )(ek2, em2t, embedding, z_flat)

    z_q = jnp.transpose(zq_flat.reshape(_B, 8, 32, 32, _D), (0, 4, 1, 2, 3))
    idx = idx2.reshape(_N)
    return (z_q, loss[0, 0], idx, perp[0, 0])


# speedup vs baseline: 1.6034x; 1.0478x over previous
import jax
import jax.numpy as jnp
from jax.experimental import pallas as pl
from jax.experimental.pallas import tpu as pltpu

_B = 4
_D = 256
_K = 1024
_THW = 8 * 32 * 32
_N = _B * _THW
_R = 2048
_G = _N // _R
_COMMIT = 0.25


def _vq_body(ek2_ref, em2t_ref, e_ref, z_ref,
             zq_ref, idx_ref, loss_ref, perp_ref,
             ssd_acc, cnt_acc):
    g = pl.program_id(0)
    first = g == 0
    last = g == _G - 1

    @pl.when(first)
    def _init():
        ssd_acc[0, 0] = 0.0
        cnt_acc[...] = jnp.zeros_like(cnt_acc)

    _H = _R // 4
    iota = jax.lax.broadcasted_iota(jnp.int32, (_H, _K), 1)
    ssd_parts = []
    cnt_parts = []
    for h in range(4):
        rows = pl.ds(h * _H, _H)
        z_blk = z_ref[rows, :]
        s2 = jnp.dot(z_blk, em2t_ref[...],
                     preferred_element_type=jnp.float32)
        zn2 = jnp.sum(z_blk * z_blk, axis=1, keepdims=True)
        d = (zn2 + ek2_ref[...]) + s2

        m = jnp.min(d, axis=1, keepdims=True)
        idx = jnp.min(jnp.where(d == m, iota, _K),
                      axis=1, keepdims=True)
        idx_ref[rows, :] = idx

        onehot = (iota == idx).astype(jnp.float32)
        zq = jnp.dot(onehot, e_ref[...],
                     preferred_element_type=jnp.float32)
        diff = zq - z_blk
        zq_ref[rows, :] = z_blk + diff
        ssd_parts.append(jnp.sum(diff * diff))
        cnt_parts.append(jnp.sum(onehot, axis=0, keepdims=True))
    ssd_acc[0, 0] += (ssd_parts[0] + ssd_parts[1]) + (ssd_parts[2] + ssd_parts[3])
    cnt_acc[...] += (cnt_parts[0] + cnt_parts[1]) + (cnt_parts[2] + cnt_parts[3])

    @pl.when(last)
    def _fini():
        loss = (1.0 + _COMMIT) * ssd_acc[0, 0] / float(_N * _D)
        loss_ref[...] = jnp.reshape(loss, (1, 1))
        p = cnt_acc[...] * (1.0 / float(_N))
        perp = jnp.exp(-jnp.sum(p * jnp.log(p + 1e-10)))
        perp_ref[...] = jnp.reshape(perp, (1, 1))


def kernel(z, embedding):
    z_flat = jnp.transpose(z, (0, 2, 3, 4, 1)).reshape(_N, _D)
    ek2 = (embedding ** 2).sum(axis=1).reshape(1, _K)
    em2t = -2.0 * embedding.T

    zq_flat, idx2, loss, perp = pl.pallas_call(
        _vq_body,
        grid=(_G,),
        in_specs=[
            pl.BlockSpec((1, _K), lambda g: (0, 0)),
            pl.BlockSpec((_D, _K), lambda g: (0, 0)),
            pl.BlockSpec((_K, _D), lambda g: (0, 0)),
            pl.BlockSpec((_R, _D), lambda g: (g, 0)),
        ],
        out_specs=[
            pl.BlockSpec((_R, _D), lambda g: (g, 0)),
            pl.BlockSpec((_R, 1), lambda g: (g, 0)),
            pl.BlockSpec((1, 1), lambda g: (0, 0)),
            pl.BlockSpec((1, 1), lambda g: (0, 0)),
        ],
        out_shape=[
            jax.ShapeDtypeStruct((_N, _D), jnp.float32),
            jax.ShapeDtypeStruct((_N, 1), jnp.int32),
            jax.ShapeDtypeStruct((1, 1), jnp.float32),
            jax.ShapeDtypeStruct((1, 1), jnp.float32),
        ],
        scratch_shapes=[
            pltpu.SMEM((1, 1), jnp.float32),
            pltpu.VMEM((1, _K), jnp.float32),
        ],
        compiler_params=pltpu.CompilerParams(
            dimension_semantics=("arbitrary",),
        ),
    )(ek2, em2t, embedding, z_flat)

    z_q = jnp.transpose(zq_flat.reshape(_B, 8, 32, 32, _D), (0, 4, 1, 2, 3))
    idx = idx2.reshape(_N)
    return (z_q, loss[0, 0], idx, perp[0, 0])


# speedup vs baseline: 1.6893x; 1.0536x over previous
import jax
import jax.numpy as jnp
from jax.experimental import pallas as pl
from jax.experimental.pallas import tpu as pltpu

_B = 4
_D = 256
_K = 1024
_THW = 8 * 32 * 32
_N = _B * _THW
_R = 4096
_G = _N // _R
_COMMIT = 0.25


def _vq_body(ek2_ref, em2t_ref, e_ref, z_ref,
             zq_ref, idx_ref, loss_ref, perp_ref,
             ssd_acc, cnt_acc):
    g = pl.program_id(0)
    first = g == 0
    last = g == _G - 1

    @pl.when(first)
    def _init():
        ssd_acc[0, 0] = 0.0
        cnt_acc[...] = jnp.zeros_like(cnt_acc)

    _H = _R // 8
    iota = jax.lax.broadcasted_iota(jnp.int32, (_H, _K), 1)
    ssd_parts = []
    cnt_parts = []
    for h in range(8):
        rows = pl.ds(h * _H, _H)
        z_blk = z_ref[rows, :]
        s2 = jnp.dot(z_blk, em2t_ref[...],
                     preferred_element_type=jnp.float32)
        zn2 = jnp.sum(z_blk * z_blk, axis=1, keepdims=True)
        d = (zn2 + ek2_ref[...]) + s2

        m = jnp.min(d, axis=1, keepdims=True)
        idx = jnp.min(jnp.where(d == m, iota, _K),
                      axis=1, keepdims=True)
        idx_ref[rows, :] = idx

        onehot = (iota == idx).astype(jnp.float32)
        zq = jnp.dot(onehot, e_ref[...],
                     preferred_element_type=jnp.float32)
        diff = zq - z_blk
        zq_ref[rows, :] = z_blk + diff
        ssd_parts.append(jnp.sum(diff * diff))
        cnt_parts.append(jnp.sum(onehot, axis=0, keepdims=True))
    ssd_acc[0, 0] += sum(ssd_parts[1:], ssd_parts[0])
    cnt_acc[...] += sum(cnt_parts[1:], cnt_parts[0])

    @pl.when(last)
    def _fini():
        loss = (1.0 + _COMMIT) * ssd_acc[0, 0] / float(_N * _D)
        loss_ref[...] = jnp.reshape(loss, (1, 1))
        p = cnt_acc[...] * (1.0 / float(_N))
        perp = jnp.exp(-jnp.sum(p * jnp.log(p + 1e-10)))
        perp_ref[...] = jnp.reshape(perp, (1, 1))


def kernel(z, embedding):
    z_flat = jnp.transpose(z, (0, 2, 3, 4, 1)).reshape(_N, _D)
    ek2 = (embedding ** 2).sum(axis=1).reshape(1, _K)
    em2t = -2.0 * embedding.T

    zq_flat, idx2, loss, perp = pl.pallas_call(
        _vq_body,
        grid=(_G,),
        in_specs=[
            pl.BlockSpec((1, _K), lambda g: (0, 0)),
            pl.BlockSpec((_D, _K), lambda g: (0, 0)),
            pl.BlockSpec((_K, _D), lambda g: (0, 0)),
            pl.BlockSpec((_R, _D), lambda g: (g, 0)),
        ],
        out_specs=[
            pl.BlockSpec((_R, _D), lambda g: (g, 0)),
            pl.BlockSpec((_R, 1), lambda g: (g, 0)),
            pl.BlockSpec((1, 1), lambda g: (0, 0)),
            pl.BlockSpec((1, 1), lambda g: (0, 0)),
        ],
        out_shape=[
            jax.ShapeDtypeStruct((_N, _D), jnp.float32),
            jax.ShapeDtypeStruct((_N, 1), jnp.int32),
            jax.ShapeDtypeStruct((1, 1), jnp.float32),
            jax.ShapeDtypeStruct((1, 1), jnp.float32),
        ],
        scratch_shapes=[
            pltpu.SMEM((1, 1), jnp.float32),
            pltpu.VMEM((1, _K), jnp.float32),
        ],
        compiler_params=pltpu.CompilerParams(
            dimension_semantics=("arbitrary",),
        ),
    )(ek2, em2t, embedding, z_flat)

    z_q = jnp.transpose(zq_flat.reshape(_B, 8, 32, 32, _D), (0, 4, 1, 2, 3))
    idx = idx2.reshape(_N)
    return (z_q, loss[0, 0], idx, perp[0, 0])


# speedup vs baseline: 1.6996x; 1.0061x over previous
import jax
import jax.numpy as jnp
from jax.experimental import pallas as pl
from jax.experimental.pallas import tpu as pltpu

_B = 4
_D = 256
_K = 1024
_THW = 8 * 32 * 32
_N = _B * _THW
_R = 8192
_G = _N // _R
_COMMIT = 0.25


def _vq_body(ek2_ref, em2t_ref, e_ref, z_ref,
             zq_ref, idx_ref, loss_ref, perp_ref,
             ssd_acc, cnt_acc):
    g = pl.program_id(0)
    first = g == 0
    last = g == _G - 1

    @pl.when(first)
    def _init():
        ssd_acc[0, 0] = 0.0
        cnt_acc[...] = jnp.zeros_like(cnt_acc)

    _H = _R // 16
    iota = jax.lax.broadcasted_iota(jnp.int32, (_H, _K), 1)
    ssd_parts = []
    cnt_parts = []
    for h in range(16):
        rows = pl.ds(h * _H, _H)
        z_blk = z_ref[rows, :]
        s2 = jnp.dot(z_blk, em2t_ref[...],
                     preferred_element_type=jnp.float32)
        zn2 = jnp.sum(z_blk * z_blk, axis=1, keepdims=True)
        d = (zn2 + ek2_ref[...]) + s2

        m = jnp.min(d, axis=1, keepdims=True)
        idx = jnp.min(jnp.where(d == m, iota, _K),
                      axis=1, keepdims=True)
        idx_ref[rows, :] = idx

        onehot = (iota == idx).astype(jnp.float32)
        zq = jnp.dot(onehot, e_ref[...],
                     preferred_element_type=jnp.float32)
        diff = zq - z_blk
        zq_ref[rows, :] = z_blk + diff
        ssd_parts.append(jnp.sum(diff * diff))
        cnt_parts.append(jnp.sum(onehot, axis=0, keepdims=True))
    ssd_acc[0, 0] += sum(ssd_parts[1:], ssd_parts[0])
    cnt_acc[...] += sum(cnt_parts[1:], cnt_parts[0])

    @pl.when(last)
    def _fini():
        loss = (1.0 + _COMMIT) * ssd_acc[0, 0] / float(_N * _D)
        loss_ref[...] = jnp.reshape(loss, (1, 1))
        p = cnt_acc[...] * (1.0 / float(_N))
        perp = jnp.exp(-jnp.sum(p * jnp.log(p + 1e-10)))
        perp_ref[...] = jnp.reshape(perp, (1, 1))


def kernel(z, embedding):
    z_flat = jnp.transpose(z, (0, 2, 3, 4, 1)).reshape(_N, _D)
    ek2 = (embedding ** 2).sum(axis=1).reshape(1, _K)
    em2t = -2.0 * embedding.T

    zq_flat, idx2, loss, perp = pl.pallas_call(
        _vq_body,
        grid=(_G,),
        in_specs=[
            pl.BlockSpec((1, _K), lambda g: (0, 0)),
            pl.BlockSpec((_D, _K), lambda g: (0, 0)),
            pl.BlockSpec((_K, _D), lambda g: (0, 0)),
            pl.BlockSpec((_R, _D), lambda g: (g, 0)),
        ],
        out_specs=[
            pl.BlockSpec((_R, _D), lambda g: (g, 0)),
            pl.BlockSpec((_R, 1), lambda g: (g, 0)),
            pl.BlockSpec((1, 1), lambda g: (0, 0)),
            pl.BlockSpec((1, 1), lambda g: (0, 0)),
        ],
        out_shape=[
            jax.ShapeDtypeStruct((_N, _D), jnp.float32),
            jax.ShapeDtypeStruct((_N, 1), jnp.int32),
            jax.ShapeDtypeStruct((1, 1), jnp.float32),
            jax.ShapeDtypeStruct((1, 1), jnp.float32),
        ],
        scratch_shapes=[
            pltpu.SMEM((1, 1), jnp.float32),
            pltpu.VMEM((1, _K), jnp.float32),
        ],
        compiler_params=pltpu.CompilerParams(
            dimension_semantics=("arbitrary",),
        ),
    )(ek2, em2t, embedding, z_flat)

    z_q = jnp.transpose(zq_flat.reshape(_B, 8, 32, 32, _D), (0, 4, 1, 2, 3))
    idx = idx2.reshape(_N)
    return (z_q, loss[0, 0], idx, perp[0, 0])
